# baseline (speedup 1.0000x reference)
import functools

import jax
import jax.numpy as jnp
from jax import lax
from jax.experimental import pallas as pl
from jax.experimental.pallas import tpu as pltpu
from jax.experimental.pallas import tpu_sc as plsc

L = 16
C = 16
D = 64
PW = D // 2
G = D // 32
VOCAB = 1000
NPOS = 16
NW = 32
ROWS = 1024 * 50
ROWS_PER_TILE = ROWS // NW
CHUNK = 320
NCHUNK = ROWS_PER_TILE // CHUNK

_HI = jnp.int32(-65536)

RPV = 128 // C
PROWS = ROWS // RPV
PCOLS = RPV * D
TBR = 128
TNB = PROWS // TBR


def _pack_word_table(w):
    v = w.shape[0]
    bits = lax.bitcast_convert_type(w.astype(jnp.bfloat16), jnp.uint16)
    bits = bits.reshape(v, G, 2, L).astype(jnp.uint32)
    words = bits[:, :, 0, :] | (bits[:, :, 1, :] << 16)
    return lax.bitcast_convert_type(words, jnp.int32).reshape(v * PW)


def _sc_body(tok_hbm, wword_hbm, out_hbm, wword_v, tok_v, out_v):
    wid = lax.axis_index("s") * 2 + lax.axis_index("c")
    base = wid * ROWS_PER_TILE

    pltpu.sync_copy(wword_hbm, wword_v)

    for chunk in range(NCHUNK):
        r0 = base + chunk * CHUNK
        pltpu.sync_copy(tok_hbm.at[pl.ds(r0 * C, CHUNK * C)], tok_v)

        @plsc.parallel_loop(0, CHUNK, unroll=8)
        def row_body(r):
            tok_row = tok_v[pl.ds(r * C, C)]
            acc = [[[None, None], [None, None]] for _ in range(G)]
            for c in range(C):
                t = tok_row[c] * PW
                s = c % 2
                for g in range(G):
                    w = wword_v[pl.ds(t + g * L, L)]
                    for h, wv in enumerate((
                            plsc.bitcast(w << 16, jnp.float32),
                            plsc.bitcast(w & _HI, jnp.float32))):
                        a = acc[g][h]
                        a[s] = wv if a[s] is None else a[s] + wv
            for g in range(G):
                for h in range(2):
                    out_v[pl.ds(r * D + g * 32 + h * L, L)] = \
                        acc[g][h][0] + acc[g][h][1]

        pltpu.sync_copy(out_v, out_hbm.at[pl.ds(r0 * D, CHUNK * D)])


def _tc_body(prow_ref, ball_ref, word_ref, out_ref):
    pos_blk = prow_ref[...]
    acc = word_ref[...]
    for i in range(NPOS):
        oh = (pos_blk == i).astype(jnp.bfloat16)
        acc = acc + lax.dot_general(
            oh, ball_ref[i],
            (((1,), (0,)), ((), ())),
            preferred_element_type=jnp.float32)
    out_ref[...] = acc


@functools.partial(jax.jit, static_argnames=())
def kernel(token_ids, position_ids, W_word, W_pos):
    B, W, _ = token_ids.shape
    tok = token_ids.reshape(ROWS * C).astype(jnp.int32)
    prow = position_ids.reshape(PROWS, 128).astype(jnp.int32)
    wword = _pack_word_table(W_word)

    sel = (jnp.arange(128)[:, None] // C == jnp.arange(PCOLS)[None, :] // D)
    ball = (sel.astype(jnp.float32)[None]
            * jnp.tile(W_pos, (1, RPV))[:, None, :]).astype(jnp.bfloat16)

    mesh = plsc.VectorSubcoreMesh(core_axis_name="c", subcore_axis_name="s")
    word_part = pl.kernel(
        _sc_body,
        out_type=jax.ShapeDtypeStruct((ROWS * D,), jnp.float32),
        mesh=mesh,
        compiler_params=pltpu.CompilerParams(needs_layout_passes=False),
        scratch_types=[
            pltpu.VMEM((VOCAB * PW,), jnp.int32),
            pltpu.VMEM((CHUNK * C,), jnp.int32),
            pltpu.VMEM((CHUNK * D,), jnp.float32),
        ],
    )(tok, wword)

    out = pl.pallas_call(
        _tc_body,
        grid=(TNB,),
        in_specs=[
            pl.BlockSpec((TBR, 128), lambda b: (b, 0)),
            pl.BlockSpec((NPOS, 128, PCOLS), lambda b: (0, 0, 0)),
            pl.BlockSpec((TBR, PCOLS), lambda b: (b, 0)),
        ],
        out_specs=pl.BlockSpec((TBR, PCOLS), lambda b: (b, 0)),
        out_shape=jax.ShapeDtypeStruct((PROWS, PCOLS), jnp.float32),
    )(prow, ball, word_part.reshape(PROWS, PCOLS))

    return out.reshape(B, W, D)


# speedup vs baseline: 1.0313x; 1.0313x over previous
import functools

import jax
import jax.numpy as jnp
from jax import lax
from jax.experimental import pallas as pl
from jax.experimental.pallas import tpu as pltpu
from jax.experimental.pallas import tpu_sc as plsc

L = 16
C = 16
D = 64
PW = D // 2
G = D // 32
VOCAB = 1000
NPOS = 16
NW = 32
ROWS = 1024 * 50
ROWS_PER_TILE = ROWS // NW
CHUNK = 320
NCHUNK = ROWS_PER_TILE // CHUNK

_HI = jnp.int32(-65536)

RPV = 128 // C
PROWS = ROWS // RPV
PCOLS = RPV * D
TBR = 128
TNB = PROWS // TBR


def _pack_word_table(w):
    v = w.shape[0]
    bits = lax.bitcast_convert_type(w.astype(jnp.bfloat16), jnp.uint16)
    bits = bits.reshape(v, G, 2, L).astype(jnp.uint32)
    words = bits[:, :, 0, :] | (bits[:, :, 1, :] << 16)
    return lax.bitcast_convert_type(words, jnp.int32).reshape(v * PW)


def _sc_body(tok_hbm, wword_hbm, out_hbm, wword_v, tok_v, out_v):
    wid = lax.axis_index("s") * 2 + lax.axis_index("c")
    base = wid * ROWS_PER_TILE

    pltpu.sync_copy(wword_hbm, wword_v)

    for chunk in range(NCHUNK):
        r0 = base + chunk * CHUNK
        pltpu.sync_copy(tok_hbm.at[pl.ds(r0 * C, CHUNK * C)], tok_v)

        @plsc.parallel_loop(0, CHUNK, unroll=8)
        def row_body(r):
            tok_row = tok_v[pl.ds(r * C, C)]
            acc = [[[None, None], [None, None]] for _ in range(G)]
            for c in range(C):
                t = tok_row[c] * PW
                s = c % 2
                for g in range(G):
                    w = wword_v[pl.ds(t + g * L, L)]
                    for h, wv in enumerate((
                            plsc.bitcast(w << 16, jnp.float32),
                            plsc.bitcast(w & _HI, jnp.float32))):
                        a = acc[g][h]
                        a[s] = wv if a[s] is None else a[s] + wv
            for g in range(G):
                for h in range(2):
                    out_v[pl.ds(r * D + g * 32 + h * L, L)] = \
                        acc[g][h][0] + acc[g][h][1]

        pltpu.sync_copy(out_v, out_hbm.at[pl.ds(r0 * D, CHUNK * D)])


def _tc_body(prow_ref, ball_ref, word_ref, out_ref):
    pos_blk = prow_ref[...]
    oh = jnp.concatenate([pos_blk == i for i in range(NPOS)],
                         axis=1).astype(jnp.bfloat16)
    out_ref[...] = word_ref[...] + lax.dot_general(
        oh, ball_ref[...],
        (((1,), (0,)), ((), ())),
        preferred_element_type=jnp.float32)


@functools.partial(jax.jit, static_argnames=())
def kernel(token_ids, position_ids, W_word, W_pos):
    B, W, _ = token_ids.shape
    tok = token_ids.reshape(ROWS * C).astype(jnp.int32)
    prow = position_ids.reshape(PROWS, 128).astype(jnp.int32)
    wword = _pack_word_table(W_word)

    sel = (jnp.arange(128)[:, None] // C == jnp.arange(PCOLS)[None, :] // D)
    ball = (sel.astype(jnp.float32)[None]
            * jnp.tile(W_pos, (1, RPV))[:, None, :]).astype(jnp.bfloat16)
    ball = ball.reshape(NPOS * 128, PCOLS)

    mesh = plsc.VectorSubcoreMesh(core_axis_name="c", subcore_axis_name="s")
    word_part = pl.kernel(
        _sc_body,
        out_type=jax.ShapeDtypeStruct((ROWS * D,), jnp.float32),
        mesh=mesh,
        compiler_params=pltpu.CompilerParams(needs_layout_passes=False),
        scratch_types=[
            pltpu.VMEM((VOCAB * PW,), jnp.int32),
            pltpu.VMEM((CHUNK * C,), jnp.int32),
            pltpu.VMEM((CHUNK * D,), jnp.float32),
        ],
    )(tok, wword)

    out = pl.pallas_call(
        _tc_body,
        grid=(TNB,),
        in_specs=[
            pl.BlockSpec((TBR, 128), lambda b: (b, 0)),
            pl.BlockSpec((NPOS * 128, PCOLS), lambda b: (0, 0)),
            pl.BlockSpec((TBR, PCOLS), lambda b: (b, 0)),
        ],
        out_specs=pl.BlockSpec((TBR, PCOLS), lambda b: (b, 0)),
        out_shape=jax.ShapeDtypeStruct((PROWS, PCOLS), jnp.float32),
    )(prow, ball, word_part.reshape(PROWS, PCOLS))

    return out.reshape(B, W, D)


# speedup vs baseline: 1.0954x; 1.0622x over previous
import functools

import jax
import jax.numpy as jnp
from jax import lax
from jax.experimental import pallas as pl
from jax.experimental.pallas import tpu as pltpu
from jax.experimental.pallas import tpu_sc as plsc

L = 16
C = 16
D = 64
PW = D // 2
G = D // 32
VOCAB = 1000
NPOS = 16
NW = 32
ROWS = 1024 * 50
ROWS_PER_TILE = ROWS // NW
CHUNK = 320
NCHUNK = ROWS_PER_TILE // CHUNK

_HI = jnp.int32(-65536)

RPV = 128 // C
PROWS = ROWS // RPV
PCOLS = RPV * D
TBR = 256
TNB = PROWS // TBR


def _pack_word_table(w):
    v = w.shape[0]
    bits = lax.bitcast_convert_type(w.astype(jnp.bfloat16), jnp.uint16)
    bits = bits.reshape(v, G, 2, L).astype(jnp.uint32)
    words = bits[:, :, 0, :] | (bits[:, :, 1, :] << 16)
    return lax.bitcast_convert_type(words, jnp.int32).reshape(v * PW)


def _sc_body(tok_hbm, wword_hbm, out_hbm, wword_v, tok_v, out_v):
    wid = lax.axis_index("s") * 2 + lax.axis_index("c")
    base = wid * ROWS_PER_TILE

    pltpu.sync_copy(wword_hbm, wword_v)

    for chunk in range(NCHUNK):
        r0 = base + chunk * CHUNK
        pltpu.sync_copy(tok_hbm.at[pl.ds(r0 * C, CHUNK * C)], tok_v)

        @plsc.parallel_loop(0, CHUNK, unroll=8)
        def row_body(r):
            tok_row = tok_v[pl.ds(r * C, C)]
            acc = [[[None, None], [None, None]] for _ in range(G)]
            for c in range(C):
                t = tok_row[c] * PW
                s = c % 2
                for g in range(G):
                    w = wword_v[pl.ds(t + g * L, L)]
                    for h, wv in enumerate((
                            plsc.bitcast(w << 16, jnp.float32),
                            plsc.bitcast(w & _HI, jnp.float32))):
                        a = acc[g][h]
                        a[s] = wv if a[s] is None else a[s] + wv
            for g in range(G):
                for h in range(2):
                    out_v[pl.ds(r * D + g * 32 + h * L, L)] = \
                        acc[g][h][0] + acc[g][h][1]

        pltpu.sync_copy(out_v, out_hbm.at[pl.ds(r0 * D, CHUNK * D)])


def _tc_body(prow_ref, ball_ref, word_ref, out_ref):
    pos_blk = prow_ref[...]
    oh = jnp.concatenate([pos_blk == i for i in range(NPOS)],
                         axis=1).astype(jnp.bfloat16)
    out_ref[...] = word_ref[...] + lax.dot_general(
        oh, ball_ref[...],
        (((1,), (0,)), ((), ())),
        preferred_element_type=jnp.float32)


@functools.partial(jax.jit, static_argnames=())
def kernel(token_ids, position_ids, W_word, W_pos):
    B, W, _ = token_ids.shape
    tok = token_ids.reshape(ROWS * C).astype(jnp.int32)
    prow = position_ids.reshape(PROWS, 128).astype(jnp.int32)
    wword = _pack_word_table(W_word)

    sel = (jnp.arange(128)[:, None] // C == jnp.arange(PCOLS)[None, :] // D)
    ball = (sel.astype(jnp.float32)[None]
            * jnp.tile(W_pos, (1, RPV))[:, None, :]).astype(jnp.bfloat16)
    ball = ball.reshape(NPOS * 128, PCOLS)

    mesh = plsc.VectorSubcoreMesh(core_axis_name="c", subcore_axis_name="s")
    word_part = pl.kernel(
        _sc_body,
        out_type=jax.ShapeDtypeStruct((ROWS * D,), jnp.float32),
        mesh=mesh,
        compiler_params=pltpu.CompilerParams(needs_layout_passes=False),
        scratch_types=[
            pltpu.VMEM((VOCAB * PW,), jnp.int32),
            pltpu.VMEM((CHUNK * C,), jnp.int32),
            pltpu.VMEM((CHUNK * D,), jnp.float32),
        ],
    )(tok, wword)

    out = pl.pallas_call(
        _tc_body,
        grid=(TNB,),
        in_specs=[
            pl.BlockSpec((TBR, 128), lambda b: (b, 0)),
            pl.BlockSpec((NPOS * 128, PCOLS), lambda b: (0, 0)),
            pl.BlockSpec((TBR, PCOLS), lambda b: (b, 0)),
        ],
        out_specs=pl.BlockSpec((TBR, PCOLS), lambda b: (b, 0)),
        out_shape=jax.ShapeDtypeStruct((PROWS, PCOLS), jnp.float32),
    )(prow, ball, word_part.reshape(PROWS, PCOLS))

    return out.reshape(B, W, D)


# speedup vs baseline: 1.1358x; 1.0368x over previous
import functools

import jax
import jax.numpy as jnp
from jax import lax
from jax.experimental import pallas as pl
from jax.experimental.pallas import tpu as pltpu
from jax.experimental.pallas import tpu_sc as plsc

L = 16
C = 16
D = 64
PW = D // 2
G = D // 32
VOCAB = 1000
NPOS = 16
NW = 32
ROWS = 1024 * 50
ROWS_PER_TILE = ROWS // NW
CHUNK = 320
NCHUNK = ROWS_PER_TILE // CHUNK

_HI = jnp.int32(-65536)

RPV = 128 // C
PROWS = ROWS // RPV
PCOLS = RPV * D
TBR = 640
TNB = PROWS // TBR


def _pack_word_table(w):
    v = w.shape[0]
    bits = lax.bitcast_convert_type(w.astype(jnp.bfloat16), jnp.uint16)
    bits = bits.reshape(v, G, 2, L).astype(jnp.uint32)
    words = bits[:, :, 0, :] | (bits[:, :, 1, :] << 16)
    return lax.bitcast_convert_type(words, jnp.int32).reshape(v * PW)


def _sc_body(tok_hbm, wword_hbm, out_hbm, wword_v, tok_v, out_v):
    wid = lax.axis_index("s") * 2 + lax.axis_index("c")
    base = wid * ROWS_PER_TILE

    pltpu.sync_copy(wword_hbm, wword_v)

    for chunk in range(NCHUNK):
        r0 = base + chunk * CHUNK
        pltpu.sync_copy(tok_hbm.at[pl.ds(r0 * C, CHUNK * C)], tok_v)

        @plsc.parallel_loop(0, CHUNK, unroll=8)
        def row_body(r):
            tok_row = tok_v[pl.ds(r * C, C)]
            acc = [[[None, None], [None, None]] for _ in range(G)]
            for c in range(C):
                t = tok_row[c] * PW
                s = c % 2
                for g in range(G):
                    w = wword_v[pl.ds(t + g * L, L)]
                    for h, wv in enumerate((
                            plsc.bitcast(w << 16, jnp.float32),
                            plsc.bitcast(w & _HI, jnp.float32))):
                        a = acc[g][h]
                        a[s] = wv if a[s] is None else a[s] + wv
            for g in range(G):
                for h in range(2):
                    out_v[pl.ds(r * D + g * 32 + h * L, L)] = \
                        acc[g][h][0] + acc[g][h][1]

        pltpu.sync_copy(out_v, out_hbm.at[pl.ds(r0 * D, CHUNK * D)])


def _tc_body(prow_ref, ball_ref, word_ref, out_ref):
    pos_blk = prow_ref[...]
    oh = jnp.concatenate([pos_blk == i for i in range(NPOS)],
                         axis=1).astype(jnp.bfloat16)
    out_ref[...] = word_ref[...] + lax.dot_general(
        oh, ball_ref[...],
        (((1,), (0,)), ((), ())),
        preferred_element_type=jnp.float32)


@functools.partial(jax.jit, static_argnames=())
def kernel(token_ids, position_ids, W_word, W_pos):
    B, W, _ = token_ids.shape
    tok = token_ids.reshape(ROWS * C).astype(jnp.int32)
    prow = position_ids.reshape(PROWS, 128).astype(jnp.int32)
    wword = _pack_word_table(W_word)

    sel = (jnp.arange(128)[:, None] // C == jnp.arange(PCOLS)[None, :] // D)
    ball = (sel.astype(jnp.float32)[None]
            * jnp.tile(W_pos, (1, RPV))[:, None, :]).astype(jnp.bfloat16)
    ball = ball.reshape(NPOS * 128, PCOLS)

    mesh = plsc.VectorSubcoreMesh(core_axis_name="c", subcore_axis_name="s")
    word_part = pl.kernel(
        _sc_body,
        out_type=jax.ShapeDtypeStruct((ROWS * D,), jnp.float32),
        mesh=mesh,
        compiler_params=pltpu.CompilerParams(needs_layout_passes=False),
        scratch_types=[
            pltpu.VMEM((VOCAB * PW,), jnp.int32),
            pltpu.VMEM((CHUNK * C,), jnp.int32),
            pltpu.VMEM((CHUNK * D,), jnp.float32),
        ],
    )(tok, wword)

    out = pl.pallas_call(
        _tc_body,
        grid=(TNB,),
        in_specs=[
            pl.BlockSpec((TBR, 128), lambda b: (b, 0)),
            pl.BlockSpec((NPOS * 128, PCOLS), lambda b: (0, 0)),
            pl.BlockSpec((TBR, PCOLS), lambda b: (b, 0)),
        ],
        out_specs=pl.BlockSpec((TBR, PCOLS), lambda b: (b, 0)),
        out_shape=jax.ShapeDtypeStruct((PROWS, PCOLS), jnp.float32),
    )(prow, ball, word_part.reshape(PROWS, PCOLS))

    return out.reshape(B, W, D)


# speedup vs baseline: 1.1391x; 1.0029x over previous
import functools

import jax
import jax.numpy as jnp
from jax import lax
from jax.experimental import pallas as pl
from jax.experimental.pallas import tpu as pltpu
from jax.experimental.pallas import tpu_sc as plsc

L = 16
C = 16
D = 64
PW = D // 2
G = D // 32
VOCAB = 1000
NPOS = 16
NW = 32
ROWS = 1024 * 50
ROWS_PER_TILE = ROWS // NW
CHUNK = 320
NCHUNK = ROWS_PER_TILE // CHUNK

_HI = jnp.int32(-65536)

RPV = 128 // C
PROWS = ROWS // RPV
PCOLS = RPV * D
TBR = 800
TNB = PROWS // TBR


def _pack_word_table(w):
    v = w.shape[0]
    bits = lax.bitcast_convert_type(w.astype(jnp.bfloat16), jnp.uint16)
    bits = bits.reshape(v, G, 2, L).astype(jnp.uint32)
    words = bits[:, :, 0, :] | (bits[:, :, 1, :] << 16)
    return lax.bitcast_convert_type(words, jnp.int32).reshape(v * PW)


def _sc_body(tok_hbm, wword_hbm, out_hbm, wword_v, tok_v, out_v):
    wid = lax.axis_index("s") * 2 + lax.axis_index("c")
    base = wid * ROWS_PER_TILE

    pltpu.sync_copy(wword_hbm, wword_v)

    for chunk in range(NCHUNK):
        r0 = base + chunk * CHUNK
        pltpu.sync_copy(tok_hbm.at[pl.ds(r0 * C, CHUNK * C)], tok_v)

        @plsc.parallel_loop(0, CHUNK, unroll=8)
        def row_body(r):
            tok_row = tok_v[pl.ds(r * C, C)]
            acc = [[[None, None], [None, None]] for _ in range(G)]
            for c in range(C):
                t = tok_row[c] * PW
                s = c % 2
                for g in range(G):
                    w = wword_v[pl.ds(t + g * L, L)]
                    for h, wv in enumerate((
                            plsc.bitcast(w << 16, jnp.float32),
                            plsc.bitcast(w & _HI, jnp.float32))):
                        a = acc[g][h]
                        a[s] = wv if a[s] is None else a[s] + wv
            for g in range(G):
                for h in range(2):
                    out_v[pl.ds(r * D + g * 32 + h * L, L)] = \
                        acc[g][h][0] + acc[g][h][1]

        pltpu.sync_copy(out_v, out_hbm.at[pl.ds(r0 * D, CHUNK * D)])


def _tc_body(prow_ref, ball_ref, word_ref, out_ref):
    pos_blk = prow_ref[...]
    oh = jnp.concatenate([pos_blk == i for i in range(NPOS)],
                         axis=1).astype(jnp.bfloat16)
    out_ref[...] = word_ref[...] + lax.dot_general(
        oh, ball_ref[...],
        (((1,), (0,)), ((), ())),
        preferred_element_type=jnp.float32)


@functools.partial(jax.jit, static_argnames=())
def kernel(token_ids, position_ids, W_word, W_pos):
    B, W, _ = token_ids.shape
    tok = token_ids.reshape(ROWS * C).astype(jnp.int32)
    prow = position_ids.reshape(PROWS, 128).astype(jnp.int32)
    wword = _pack_word_table(W_word)

    sel = (jnp.arange(128)[:, None] // C == jnp.arange(PCOLS)[None, :] // D)
    ball = (sel.astype(jnp.float32)[None]
            * jnp.tile(W_pos, (1, RPV))[:, None, :]).astype(jnp.bfloat16)
    ball = ball.reshape(NPOS * 128, PCOLS)

    mesh = plsc.VectorSubcoreMesh(core_axis_name="c", subcore_axis_name="s")
    word_part = pl.kernel(
        _sc_body,
        out_type=jax.ShapeDtypeStruct((ROWS * D,), jnp.float32),
        mesh=mesh,
        compiler_params=pltpu.CompilerParams(needs_layout_passes=False),
        scratch_types=[
            pltpu.VMEM((VOCAB * PW,), jnp.int32),
            pltpu.VMEM((CHUNK * C,), jnp.int32),
            pltpu.VMEM((CHUNK * D,), jnp.float32),
        ],
    )(tok, wword)

    out = pl.pallas_call(
        _tc_body,
        grid=(TNB,),
        in_specs=[
            pl.BlockSpec((TBR, 128), lambda b: (b, 0)),
            pl.BlockSpec((NPOS * 128, PCOLS), lambda b: (0, 0)),
            pl.BlockSpec((TBR, PCOLS), lambda b: (b, 0)),
        ],
        out_specs=pl.BlockSpec((TBR, PCOLS), lambda b: (b, 0)),
        out_shape=jax.ShapeDtypeStruct((PROWS, PCOLS), jnp.float32),
    )(prow, ball, word_part.reshape(PROWS, PCOLS))

    return out.reshape(B, W, D)
